# Initial kernel scaffold; baseline (speedup 1.0000x reference)
import jax
import jax.numpy as jnp
from jax.experimental import pallas as pl


def kernel(x, target_token, wq, bq, wk, bk, wv, bv, gather_w, gather_b, wo, bo):
    raise NotImplementedError("write your pallas kernel here")


# speedup vs baseline: 4.5712x; 4.5712x over previous
import jax
import jax.numpy as jnp
from jax.experimental import pallas as pl

B, L, D = 2, 2048, 768
H = 12
DH = D // H
GROUPS = 128
NPG = 16
SCALE = float(DH) ** -0.5

_DN = (((1,), (1,)), ((), ()))
_HI = jax.lax.Precision.HIGHEST


def _dot_serial256(a, b):
    kdim = a.shape[1]
    acc = None
    for lo in range(0, kdim, 256):
        c = jax.lax.dot_general(a[:, lo:lo + 256], b[:, lo:lo + 256], _DN,
                                preferred_element_type=jnp.float32)
        acc = c if acc is None else acc + c
    return acc


def _fused_body(x_ref, tt_ref, wq_ref, bq_ref, wk_ref, bk_ref, wv_ref,
                bv_ref, gw_ref, gb_ref, wo_ref, bo_ref, out_ref):
    h = pl.program_id(1)
    x = x_ref[0]

    k = _dot_serial256(x, wk_ref[...]) + bk_ref[0]
    v = _dot_serial256(x, wv_ref[...]) + bv_ref[0]
    q = _dot_serial256(tt_ref[...], wq_ref[...]) + bq_ref[0]

    s = jax.lax.dot_general(q, k, _DN,
                            preferred_element_type=jnp.float32) * SCALE

    m0 = jnp.max(s, axis=1, keepdims=True)
    u = jnp.exp(s - m0)
    s = u / jnp.sum(u, axis=1, keepdims=True)

    colidx = jax.lax.broadcasted_iota(jnp.int32, (GROUPS, L), 1)
    gw = gw_ref[...]
    w_sel = jnp.zeros((GROUPS, L), jnp.float32)
    for t in range(NPG):
        m = jnp.max(s, axis=1, keepdims=True)
        cand = jnp.where(s == m, colidx, L)
        first = jnp.min(cand, axis=1, keepdims=True)
        onehot = cand == first
        w_sel = w_sel + jnp.where(onehot, gw[:, t:t + 1], 0.0)
        s = jnp.where(onehot, -jnp.inf, s)

    xo = jnp.dot(w_sel, v, preferred_element_type=jnp.float32,
                 precision=_HI)
    xo = xo + gb_ref[...]

    contrib = jnp.dot(xo, wo_ref[0], preferred_element_type=jnp.float32,
                      precision=_HI)

    @pl.when(h == 0)
    def _():
        out_ref[0] = contrib + bo_ref[...]

    @pl.when(h != 0)
    def _():
        out_ref[0] = out_ref[0] + contrib


@jax.jit
def kernel(x, target_token, wq, bq, wk, bk, wv, bv, gather_w, gather_b,
           wo, bo):
    bq2 = bq.reshape(H, 1, DH)
    bk2 = bk.reshape(H, 1, DH)
    bv2 = bv.reshape(H, 1, DH)
    gb2 = gather_b.reshape(GROUPS, 1)
    bo2 = bo.reshape(1, D)
    wo_t = wo.T.reshape(H, DH, D)

    grid = (B, H)
    out = pl.pallas_call(
        _fused_body,
        grid=grid,
        in_specs=[
            pl.BlockSpec((1, L, D), lambda b, h: (b, 0, 0)),
            pl.BlockSpec((GROUPS, D), lambda b, h: (0, 0)),
            pl.BlockSpec((DH, D), lambda b, h: (h, 0)),
            pl.BlockSpec((1, 1, DH), lambda b, h: (h, 0, 0)),
            pl.BlockSpec((DH, D), lambda b, h: (h, 0)),
            pl.BlockSpec((1, 1, DH), lambda b, h: (h, 0, 0)),
            pl.BlockSpec((DH, D), lambda b, h: (h, 0)),
            pl.BlockSpec((1, 1, DH), lambda b, h: (h, 0, 0)),
            pl.BlockSpec((GROUPS, NPG), lambda b, h: (0, 0)),
            pl.BlockSpec((GROUPS, 1), lambda b, h: (0, 0)),
            pl.BlockSpec((1, DH, D), lambda b, h: (h, 0, 0)),
            pl.BlockSpec((1, D), lambda b, h: (0, 0)),
        ],
        out_specs=pl.BlockSpec((1, GROUPS, D), lambda b, h: (b, 0, 0)),
        out_shape=jax.ShapeDtypeStruct((B, GROUPS, D), jnp.float32),
    )(x, target_token, wq, bq2, wk, bk2, wv, bv2, gather_w, gb2, wo_t, bo2)
    return out
